# Initial kernel scaffold; baseline (speedup 1.0000x reference)
import jax
import jax.numpy as jnp
from jax.experimental import pallas as pl


def kernel(x, edge_index, edge_attr, Wl, We, att, bias):
    raise NotImplementedError("write your pallas kernel here")


# speedup vs baseline: 11.7609x; 11.7609x over previous
import dataclasses
import functools

import jax
import jax.numpy as jnp
from jax import lax
from jax.experimental import pallas as pl
from jax.experimental.pallas import tpu as pltpu
from jax.experimental.pallas import tpu_sc as plsc

N = 10000
E = 320000
C = 128
EDGE_DIM = 16
SLOPE = 0.2

NC = 2
NS = 16
NW = NC * NS
EPT = E // NW
CHUNK = 80
NCHUNK = EPT // CHUNK
BLKCH = 25
NBLKE = NCHUNK // BLKCH
EPB = BLKCH * CHUNK
RPS = 624
ZROWS = 16

NBLK = 1000
EBLK = 8000


def _node_tc(x_ref, wl_ref, att_ref, xh_ref, ai_ref, aj_ref, mxa_ref):
    xh = jnp.dot(x_ref[...], wl_ref[...], preferred_element_type=jnp.float32)
    xh_ref[...] = xh
    att1 = att_ref[0, 0:C]
    att2 = att_ref[0, C:2 * C]
    a_i = jnp.sum(xh * att1[None, :], axis=1, keepdims=True)
    a_j = jnp.sum(xh * att2[None, :], axis=1, keepdims=True)
    ai_ref[...] = a_i
    aj_ref[...] = a_j

    @pl.when(pl.program_id(0) == 0)
    def _():
        mxa_ref[...] = jnp.full((1, 2), -jnp.inf, jnp.float32)

    cur = jnp.concatenate(
        [jnp.max(a_i).reshape(1, 1), jnp.max(a_j).reshape(1, 1)], axis=1)
    mxa_ref[...] = jnp.maximum(mxa_ref[...], cur)


def _edge_tc(ea_ref, we_ref, att_ref, b_ref, mxb_ref):
    att3 = att_ref[0, 2 * C:3 * C]
    wa = jnp.sum(we_ref[...] * att3[None, :], axis=1)
    b = jnp.sum(ea_ref[...] * wa[None, :], axis=1, keepdims=True)
    b_ref[...] = b

    @pl.when(pl.program_id(0) == 0)
    def _():
        mxb_ref[...] = jnp.full((1, 1), -jnp.inf, jnp.float32)

    mxb_ref[...] = jnp.maximum(mxb_ref[...], jnp.max(b).reshape(1, 1))


def _ssum_tc(s_ref, o_ref):
    o_ref[...] = jnp.sum(s_ref[:, 0, :], axis=0, keepdims=True)


def _final_tc(v_ref, s_ref, bias_ref, o_ref):
    v = v_ref[0] + v_ref[1]
    o_ref[...] = v / (s_ref[...] + 1e-16) + bias_ref[...]


def _exp_tc(al_ref, mxa_ref, mxb_ref, w_ref):
    m_raw = mxa_ref[0, 0] + mxa_ref[0, 1] + mxb_ref[0, 0]
    m = jnp.where(m_raw >= 0, m_raw, m_raw * SLOPE)
    w_ref[...] = jnp.exp(al_ref[...] - m)


def _sc_compiler_params():
  cp = pltpu.CompilerParams()
  if "needs_layout_passes" in pltpu.CompilerParams.__dataclass_fields__:
    cp = dataclasses.replace(cp, needs_layout_passes=False)
  return cp


@functools.cache
def _make_sc_alpha():
  mesh = plsc.VectorSubcoreMesh(core_axis_name="c", subcore_axis_name="s")

  @functools.partial(
    pl.kernel,
    compiler_params=_sc_compiler_params(),
    out_type=jax.ShapeDtypeStruct((NW, NBLKE, 1, EPB), jnp.float32),
    mesh=mesh,
    scratch_types=[
        pltpu.VMEM((N,), jnp.float32),
        pltpu.VMEM((N,), jnp.float32),
        pltpu.VMEM((1, EPB), jnp.int32),
        pltpu.VMEM((1, EPB), jnp.int32),
        pltpu.VMEM((1, EPB), jnp.float32),
        pltpu.VMEM((1, EPB), jnp.float32),
        pltpu.VMEM((16,), jnp.float32),
    ],
  )
  def _sc_alpha(row_hbm, col_hbm, b_hbm, ai_hbm, aj_hbm, w_hbm,
                ai_v, aj_v, rowb, colb, bb, wb, mx_v):
    c = lax.axis_index("c")
    s = lax.axis_index("s")
    g = c * NS + s

    zeros = jnp.zeros((16,), jnp.int32)
    lanes16 = lax.iota(jnp.int32, 16)

    ACP = 2000

    @pl.loop(0, N // ACP)
    def _(k):
        pltpu.sync_copy(ai_hbm.at[pl.ds(k * ACP, ACP)],
                        ai_v.at[pl.ds(k * ACP, ACP)])
        pltpu.sync_copy(aj_hbm.at[pl.ds(k * ACP, ACP)],
                        aj_v.at[pl.ds(k * ACP, ACP)])

    @pl.loop(0, NBLKE)
    def _(blk):
        pltpu.sync_copy(row_hbm.at[g, blk], rowb)
        pltpu.sync_copy(col_hbm.at[g, blk], colb)
        pltpu.sync_copy(b_hbm.at[g, blk], bb)

        @pl.loop(0, EPB // 16)
        def _(u):
            pos = lanes16 + u * 16
            zr = jnp.zeros((16,), jnp.int32)
            ir = plsc.load_gather(rowb, [zr, pos])
            ic = plsc.load_gather(colb, [zr, pos])
            va = plsc.load_gather(ai_v, [ir])
            vb = plsc.load_gather(aj_v, [ic])
            al = va + vb + plsc.load_gather(bb, [zr, pos])
            al = jnp.where(al >= 0, al, al * SLOPE)
            plsc.store_scatter(wb, [zr, pos], al)

        pltpu.sync_copy(wb, w_hbm.at[g, blk])

  return _sc_alpha


@functools.cache
def _make_sc_agg():
  mesh = plsc.VectorSubcoreMesh(core_axis_name="c", subcore_axis_name="s")

  @functools.partial(
    pl.kernel,
    compiler_params=_sc_compiler_params(),
    out_type=[
        jax.ShapeDtypeStruct((NC, N, C), jnp.float32),
        jax.ShapeDtypeStruct((NW, 1, N), jnp.float32),
    ],
    mesh=mesh,
    scratch_types=[
        pltpu.VMEM_SHARED((N, C), jnp.float32),
        pltpu.VMEM((1, N), jnp.float32),
        pltpu.VMEM((BLKCH, CHUNK), jnp.int32),
        pltpu.VMEM((BLKCH, CHUNK), jnp.int32),
        pltpu.VMEM((1, EPB), jnp.int32),
        pltpu.VMEM((1, EPB), jnp.float32),
        pltpu.VMEM((CHUNK, C), jnp.float32),
        pltpu.VMEM((ZROWS, C), jnp.float32),
        pltpu.VMEM((16,), jnp.int32),
        pltpu.SemaphoreType.DMA,
    ],
  )
  def _sc_agg(row_hbm, rowf_hbm, col_hbm, w_hbm, xh_hbm,
              vout_hbm, sout_hbm,
              accv, s_acc, rowb, colb, rowf, wvb, grows, zbv, idx_v, sem):
    c = lax.axis_index("c")
    s = lax.axis_index("s")
    g = c * NS + s

    zv16 = jnp.zeros((16,), jnp.float32)
    lanes16 = lax.iota(jnp.int32, 16)

    @pl.loop(0, ZROWS)
    def _(i):
        for j in range(C // 16):
            zbv[i, pl.ds(j * 16, 16)] = zv16

    zeros16 = jnp.zeros((16,), jnp.int32)

    @pl.loop(0, N // 16)
    def _(i):
        plsc.store_scatter(s_acc, [zeros16, lanes16 + i * 16], zv16)

    def _zero_rows(base, nrows):
        @pl.loop(0, nrows // ZROWS)
        def _(k):
            idx_v[...] = lanes16 + (base + k * ZROWS)
            pltpu.sync_copy(zbv, accv.at[idx_v])

    _zero_rows(s * RPS, RPS)

    @pl.when(s == NS - 1)
    def _():
        _zero_rows(NS * RPS, N - NS * RPS)

    plsc.subcore_barrier()

    @pl.loop(0, NBLKE)
    def _(blk):
        pltpu.sync_copy(row_hbm.at[g, blk], rowb)
        pltpu.sync_copy(rowf_hbm.at[g, blk], rowf)
        pltpu.sync_copy(col_hbm.at[g, blk], colb)
        pltpu.sync_copy(w_hbm.at[g, blk], wvb)

        @pl.loop(0, EPB // 16)
        def _(u):
            pos = lanes16 + u * 16
            wv = plsc.load_gather(wvb, [zeros16, pos])
            ir = plsc.load_gather(rowf, [zeros16, pos])
            plsc.addupdate_scatter(s_acc, [zeros16, ir], wv)

        @pl.loop(0, BLKCH)
        def _(t):
            pltpu.async_copy(xh_hbm.at[colb.at[t]], grows, sem).wait()

            @pl.loop(0, CHUNK)
            def _(e):
                eidx = jnp.zeros((16,), jnp.int32) + (t * CHUNK + e)
                wsplat = plsc.load_gather(wvb, [zeros16, eidx])
                for j in range(C // 16):
                    grows[e, pl.ds(j * 16, 16)] = (
                        grows[e, pl.ds(j * 16, 16)] * wsplat)

            pltpu.sync_copy(grows, accv.at[rowb.at[t]], add=True)

    plsc.subcore_barrier()

    def _write_rows(base, nrows):
        @pl.loop(0, nrows // ZROWS)
        def _(k):
            b0 = base + k * ZROWS
            idx_v[...] = lanes16 + b0
            pltpu.sync_copy(accv.at[idx_v], zbv)
            pltpu.sync_copy(zbv, vout_hbm.at[c, pl.ds(b0, ZROWS)])

    _write_rows(s * RPS, RPS)

    @pl.when(s == NS - 1)
    def _():
        _write_rows(NS * RPS, N - NS * RPS)

    pltpu.sync_copy(s_acc, sout_hbm.at[g])

  return _sc_agg


@jax.jit
def kernel(x, edge_index, edge_attr, Wl, We, att, bias):
    att2d = att.reshape(1, 3 * C)

    xh, a_i, a_j, mxa = pl.pallas_call(
        _node_tc,
        grid=(N // NBLK,),
        in_specs=[
            pl.BlockSpec((NBLK, C), lambda i: (i, 0)),
            pl.BlockSpec((C, C), lambda i: (0, 0)),
            pl.BlockSpec((1, 3 * C), lambda i: (0, 0)),
        ],
        out_specs=[
            pl.BlockSpec((NBLK, C), lambda i: (i, 0)),
            pl.BlockSpec((NBLK, 1), lambda i: (i, 0)),
            pl.BlockSpec((NBLK, 1), lambda i: (i, 0)),
            pl.BlockSpec((1, 2), lambda i: (0, 0)),
        ],
        out_shape=[
            jax.ShapeDtypeStruct((N, C), jnp.float32),
            jax.ShapeDtypeStruct((N, 1), jnp.float32),
            jax.ShapeDtypeStruct((N, 1), jnp.float32),
            jax.ShapeDtypeStruct((1, 2), jnp.float32),
        ],
    )(x, Wl, att2d)

    b, mxb = pl.pallas_call(
        _edge_tc,
        grid=(E // EBLK,),
        in_specs=[
            pl.BlockSpec((EBLK, EDGE_DIM), lambda i: (i, 0)),
            pl.BlockSpec((EDGE_DIM, C), lambda i: (0, 0)),
            pl.BlockSpec((1, 3 * C), lambda i: (0, 0)),
        ],
        out_specs=[
            pl.BlockSpec((EBLK, 1), lambda i: (i, 0)),
            pl.BlockSpec((1, 1), lambda i: (0, 0)),
        ],
        out_shape=[
            jax.ShapeDtypeStruct((E, 1), jnp.float32),
            jax.ShapeDtypeStruct((1, 1), jnp.float32),
        ],
    )(edge_attr, We, att2d)

    row4 = edge_index[0].reshape(NW, NBLKE, BLKCH, CHUNK)
    col4 = edge_index[1].reshape(NW, NBLKE, BLKCH, CHUNK)
    row3 = edge_index[0].reshape(NW, NBLKE, 1, EPB)
    col3 = edge_index[1].reshape(NW, NBLKE, 1, EPB)
    b3 = b.reshape(NW, NBLKE, 1, EPB)
    al3 = _make_sc_alpha()(
        row3, col3, b3, a_i.reshape(N), a_j.reshape(N))

    w2d = pl.pallas_call(
        _exp_tc,
        grid=(1,),
        in_specs=[
            pl.BlockSpec((E // 128, 128), lambda i: (0, 0)),
            pl.BlockSpec((1, 2), lambda i: (0, 0)),
            pl.BlockSpec((1, 1), lambda i: (0, 0)),
        ],
        out_specs=pl.BlockSpec((E // 128, 128), lambda i: (0, 0)),
        out_shape=jax.ShapeDtypeStruct((E // 128, 128), jnp.float32),
    )(al3.reshape(E // 128, 128), mxa, mxb)

    w3 = w2d.reshape(NW, NBLKE, 1, EPB)
    vout, sout = _make_sc_agg()(row4, row3, col4, w3, xh)

    ssum = pl.pallas_call(
        _ssum_tc,
        grid=(1,),
        in_specs=[pl.BlockSpec((NW, 1, N), lambda i: (0, 0, 0))],
        out_specs=pl.BlockSpec((1, N), lambda i: (0, 0)),
        out_shape=jax.ShapeDtypeStruct((1, N), jnp.float32),
    )(sout)

    out = pl.pallas_call(
        _final_tc,
        grid=(N // NBLK,),
        in_specs=[
            pl.BlockSpec((2, NBLK, C), lambda i: (0, i, 0)),
            pl.BlockSpec((NBLK, 1), lambda i: (i, 0)),
            pl.BlockSpec((1, C), lambda i: (0, 0)),
        ],
        out_specs=pl.BlockSpec((NBLK, C), lambda i: (i, 0)),
        out_shape=jax.ShapeDtypeStruct((N, C), jnp.float32),
    )(vout, ssum.reshape(N, 1), bias.reshape(1, C))

    return out
